# Initial kernel scaffold; baseline (speedup 1.0000x reference)
import functools

import jax
import jax.numpy as jnp
from jax import lax
from jax.experimental import pallas as pl
from jax.experimental.pallas import tpu as pltpu
from jax.experimental.pallas import tpu_sc as plsc

N = 10000
E = 320000
D = 128
H = 64
C = 40

NC = 2
NS = 16
NW = NC * NS

NP = 10240
NPT = NP // NS
CHN = 128
CPW = 79
EPW = CHN * CPW
EP = EPW * NW
C48 = 48

_mesh = plsc.VectorSubcoreMesh(core_axis_name="c", subcore_axis_name="s")


def _zero_rows(zb, width):
    rows = zb.shape[0]
    zv = jnp.zeros((16,), jnp.float32)

    @pl.loop(0, rows)
    def _(r):
        for j in range(width // 16):
            zb[r, pl.ds(j * 16, 16)] = zv


@functools.partial(
    pl.kernel,
    out_type=jax.ShapeDtypeStruct((NC, 2, NP), jnp.float32),
    mesh=_mesh,
    scratch_types=[
        pltpu.VMEM((CHN,), jnp.int32),
        pltpu.VMEM((CHN,), jnp.float32),
        pltpu.VMEM((CHN,), jnp.float32),
        pltpu.VMEM((NPT,), jnp.float32),
        pltpu.VMEM_SHARED((NP,), jnp.float32),
        pltpu.VMEM_SHARED((NP,), jnp.float32),
    ],
)
def _sc_degrees(col_hbm, w_hbm, out_hbm, colbuf, wbuf, ones, zb, degw, degc):
    cid = lax.axis_index("c")
    sid = lax.axis_index("s")
    wid = cid * NS + sid

    ov = jnp.ones((16,), jnp.float32)
    zv = jnp.zeros((16,), jnp.float32)
    for i in range(CHN // 16):
        ones[pl.ds(i * 16, 16)] = ov

    @pl.loop(0, NPT // 16)
    def _(i):
        zb[pl.ds(i * 16, 16)] = zv

    my_rows = pl.ds(sid * NPT, NPT)
    pltpu.sync_copy(zb, degw.at[my_rows])
    pltpu.sync_copy(zb, degc.at[my_rows])
    plsc.subcore_barrier()

    @pl.loop(0, CPW)
    def _(i):
        base = pl.multiple_of(wid * EPW + i * CHN, CHN)
        pltpu.sync_copy(col_hbm.at[pl.ds(base, CHN)], colbuf)
        pltpu.sync_copy(w_hbm.at[pl.ds(base, CHN)], wbuf)
        pltpu.sync_copy(wbuf, degw.at[colbuf], add=True)
        pltpu.sync_copy(ones, degc.at[colbuf], add=True)

    plsc.subcore_barrier()
    pltpu.sync_copy(degw.at[my_rows], out_hbm.at[cid, 0, my_rows])
    pltpu.sync_copy(degc.at[my_rows], out_hbm.at[cid, 1, my_rows])


@functools.partial(
    pl.kernel,
    out_type=jax.ShapeDtypeStruct((NC, NP, H), jnp.float32),
    mesh=_mesh,
    scratch_types=[
        pltpu.VMEM((CHN,), jnp.int32),
        pltpu.VMEM((CHN,), jnp.int32),
        pltpu.VMEM((CHN,), jnp.float32),
        pltpu.VMEM((CHN, H), jnp.float32),
        pltpu.VMEM((NPT // 4, H), jnp.float32),
        pltpu.VMEM_SHARED((NP, H), jnp.float32),
        pltpu.SemaphoreType.DMA,
    ],
)
def _sc_agg1(row_hbm, col_hbm, w_hbm, x1_hbm, out_hbm,
             rowbuf, colbuf, wbuf, msg, zb, acc, sem):
    cid = lax.axis_index("c")
    sid = lax.axis_index("s")
    wid = cid * NS + sid

    _zero_rows(zb, H)
    for kk in range(4):
        pltpu.sync_copy(zb, acc.at[pl.ds(sid * NPT + kk * (NPT // 4), NPT // 4)])
    plsc.subcore_barrier()

    @pl.loop(0, CPW)
    def _(i):
        base = pl.multiple_of(wid * EPW + i * CHN, CHN)
        pltpu.sync_copy(row_hbm.at[pl.ds(base, CHN)], rowbuf)
        pltpu.sync_copy(col_hbm.at[pl.ds(base, CHN)], colbuf)
        pltpu.sync_copy(w_hbm.at[pl.ds(base, CHN)], wbuf)
        pltpu.async_copy(x1_hbm.at[rowbuf], msg, sem).wait()

        @pl.loop(0, CHN)
        def _(k):
            wv = plsc.load_gather(wbuf, [lax.broadcast_in_dim(k, (16,), ())])
            for j in range(H // 16):
                sl = (k, pl.ds(j * 16, 16))
                msg[sl] = msg[sl] * wv

        pltpu.sync_copy(msg, acc.at[colbuf], add=True)

    plsc.subcore_barrier()
    my_rows = pl.ds(sid * NPT, NPT)
    pltpu.sync_copy(acc.at[my_rows], out_hbm.at[cid, my_rows])


@functools.partial(
    pl.kernel,
    out_type=jax.ShapeDtypeStruct((NC, NP, C48), jnp.float32),
    mesh=_mesh,
    scratch_types=[
        pltpu.VMEM((CHN,), jnp.int32),
        pltpu.VMEM((CHN,), jnp.int32),
        pltpu.VMEM((CHN, C48), jnp.float32),
        pltpu.VMEM((NPT // 4, C48), jnp.float32),
        pltpu.VMEM_SHARED((NP, C48), jnp.float32),
        pltpu.SemaphoreType.DMA,
    ],
)
def _sc_agg2(row_hbm, col_hbm, t_hbm, out_hbm,
             rowbuf, colbuf, msg, zb, acc, sem):
    cid = lax.axis_index("c")
    sid = lax.axis_index("s")
    wid = cid * NS + sid

    _zero_rows(zb, C48)
    for kk in range(4):
        pltpu.sync_copy(zb, acc.at[pl.ds(sid * NPT + kk * (NPT // 4), NPT // 4)])
    plsc.subcore_barrier()

    @pl.loop(0, CPW)
    def _(i):
        base = pl.multiple_of(wid * EPW + i * CHN, CHN)
        pltpu.sync_copy(row_hbm.at[pl.ds(base, CHN)], rowbuf)
        pltpu.sync_copy(col_hbm.at[pl.ds(base, CHN)], colbuf)
        pltpu.async_copy(t_hbm.at[rowbuf], msg, sem).wait()
        pltpu.sync_copy(msg, acc.at[colbuf], add=True)

    plsc.subcore_barrier()
    my_rows = pl.ds(sid * NPT, NPT)
    pltpu.sync_copy(acc.at[my_rows], out_hbm.at[cid, my_rows])


def _tc_x1_body(x_ref, w1_ref, deg1_ref, deg2_ref, x1_ref, d1_ref, d2_ref):
    d1 = lax.rsqrt(deg1_ref[...])
    d2 = lax.rsqrt(deg2_ref[...])
    xw = jnp.dot(x_ref[...], w1_ref[...], preferred_element_type=jnp.float32)
    x1_ref[...] = xw * d1
    d1_ref[...] = d1
    d2_ref[...] = d2


_tc_x1 = pl.pallas_call(
    _tc_x1_body,
    out_shape=[
        jax.ShapeDtypeStruct((NP, H), jnp.float32),
        jax.ShapeDtypeStruct((NP, 1), jnp.float32),
        jax.ShapeDtypeStruct((NP, 1), jnp.float32),
    ],
)


def _tc_mid_body(accp_ref, x1_ref, d1_ref, d2_ref, b1_ref, w2_ref, t_ref):
    a = accp_ref[0] + accp_ref[1] + x1_ref[...]
    h = jnp.maximum(d1_ref[...] * a + b1_ref[...], 0.0)
    t_ref[...] = jnp.dot(h, w2_ref[...], preferred_element_type=jnp.float32) * d2_ref[...]


_tc_mid = pl.pallas_call(
    _tc_mid_body,
    out_shape=jax.ShapeDtypeStruct((NP, C48), jnp.float32),
)


def _tc_out_body(accp_ref, t_ref, d2_ref, b2_ref, o_ref):
    lg = d2_ref[...] * (accp_ref[0] + accp_ref[1] + t_ref[...]) + b2_ref[...]
    mask = lax.broadcasted_iota(jnp.int32, (NP, C48), 1) < C
    l = jnp.where(mask, lg, -1e30)
    mx = jnp.max(l, axis=1, keepdims=True)
    s = jnp.sum(jnp.exp(l - mx), axis=1, keepdims=True)
    o_ref[...] = l - mx - jnp.log(s)


_tc_out = pl.pallas_call(
    _tc_out_body,
    out_shape=jax.ShapeDtypeStruct((NP, C48), jnp.float32),
)


def kernel(x, edge_index, edge_weight, W1, b1, W2, b2):
    f32 = jnp.float32
    row = edge_index[0]
    col = edge_index[1]

    npad = EP - E
    pad_dst = (N + (jnp.arange(npad, dtype=jnp.int32) % (NP - N))).astype(jnp.int32)
    rowp = jnp.concatenate([row, jnp.zeros((npad,), jnp.int32)])
    colp = jnp.concatenate([col, pad_dst])
    wp = jnp.concatenate([edge_weight.astype(f32), jnp.zeros((npad,), f32)])

    xp = jnp.zeros((NP, D), f32).at[:N].set(x.astype(f32))
    w2p = jnp.zeros((H, C48), f32).at[:, :C].set(W2.astype(f32))
    b2p = jnp.zeros((1, C48), f32).at[0, :C].set(b2.astype(f32))
    b1r = b1.astype(f32).reshape(1, H)

    degs = _sc_degrees(colp, wp)
    deg1 = (degs[0, 0] + degs[1, 0] + 1.0)[:, None]
    deg2 = (degs[0, 1] + degs[1, 1] + 1.0)[:, None]

    x1, d1, d2 = _tc_x1(xp, W1.astype(f32), deg1, deg2)
    acc1 = _sc_agg1(rowp, colp, wp, x1)
    t = _tc_mid(acc1, x1, d1, d2, b1r, w2p)
    acc2 = _sc_agg2(rowp, colp, t)
    o = _tc_out(acc2, t, d2, b2p)
    return o[:N, :C]


# speedup vs baseline: 15.7192x; 15.7192x over previous
import functools

import jax
import jax.numpy as jnp
from jax import lax
from jax.experimental import pallas as pl
from jax.experimental.pallas import tpu as pltpu
from jax.experimental.pallas import tpu_sc as plsc

N = 10000
E = 320000
D = 128
H = 64
C = 40

NC = 2
NS = 16
NW = NC * NS

NP = 10240
NPT = NP // NS
CHN = 128
CPW = 79
EPW = CHN * CPW
EP = EPW * NW
C48 = 48

_mesh = plsc.VectorSubcoreMesh(core_axis_name="c", subcore_axis_name="s")


def _zero_rows(zb, width):
    rows = zb.shape[0]
    zv = jnp.zeros((16,), jnp.float32)

    @pl.loop(0, rows)
    def _(r):
        for j in range(width // 16):
            zb[r, pl.ds(j * 16, 16)] = zv


@functools.partial(
    pl.kernel,
    out_type=jax.ShapeDtypeStruct((NC, 2, NP), jnp.float32),
    mesh=_mesh,
    compiler_params=pltpu.CompilerParams(use_tc_tiling_on_sc=False),
    scratch_types=[
        pltpu.VMEM((CHN,), jnp.int32),
        pltpu.VMEM((CHN,), jnp.float32),
        pltpu.VMEM((CHN,), jnp.float32),
        pltpu.VMEM((NPT,), jnp.float32),
        pltpu.VMEM_SHARED((NP,), jnp.float32),
        pltpu.VMEM_SHARED((NP,), jnp.float32),
    ],
)
def _sc_degrees(col_hbm, w_hbm, out_hbm, colbuf, wbuf, ones, zb, degw, degc):
    cid = lax.axis_index("c")
    sid = lax.axis_index("s")
    wid = cid * NS + sid

    ov = jnp.ones((16,), jnp.float32)
    zv = jnp.zeros((16,), jnp.float32)
    for i in range(CHN // 16):
        ones[pl.ds(i * 16, 16)] = ov

    @pl.loop(0, NPT // 16)
    def _(i):
        zb[pl.ds(i * 16, 16)] = zv

    my_rows = pl.ds(sid * NPT, NPT)
    pltpu.sync_copy(zb, degw.at[my_rows])
    pltpu.sync_copy(zb, degc.at[my_rows])
    plsc.subcore_barrier()

    @pl.loop(0, CPW)
    def _(i):
        base = pl.multiple_of(wid * EPW + i * CHN, CHN)
        pltpu.sync_copy(col_hbm.at[pl.ds(base, CHN)], colbuf)
        pltpu.sync_copy(w_hbm.at[pl.ds(base, CHN)], wbuf)
        pltpu.sync_copy(wbuf, degw.at[colbuf], add=True)
        pltpu.sync_copy(ones, degc.at[colbuf], add=True)

    plsc.subcore_barrier()
    pltpu.sync_copy(degw.at[my_rows], out_hbm.at[cid, 0, my_rows])
    pltpu.sync_copy(degc.at[my_rows], out_hbm.at[cid, 1, my_rows])


@functools.partial(
    pl.kernel,
    out_type=jax.ShapeDtypeStruct((NC, NP, H), jnp.float32),
    mesh=_mesh,
    compiler_params=pltpu.CompilerParams(use_tc_tiling_on_sc=False),
    scratch_types=[
        pltpu.VMEM((CHN,), jnp.int32),
        pltpu.VMEM((CHN,), jnp.int32),
        pltpu.VMEM((CHN,), jnp.float32),
        pltpu.VMEM((CHN, H), jnp.float32),
        pltpu.VMEM((NPT // 4, H), jnp.float32),
        pltpu.VMEM_SHARED((NP, H), jnp.float32),
        pltpu.SemaphoreType.DMA,
    ],
)
def _sc_agg1(row_hbm, col_hbm, w_hbm, x1_hbm, out_hbm,
             rowbuf, colbuf, wbuf, msg, zb, acc, sem):
    cid = lax.axis_index("c")
    sid = lax.axis_index("s")
    wid = cid * NS + sid

    _zero_rows(zb, H)
    for kk in range(4):
        pltpu.sync_copy(zb, acc.at[pl.ds(sid * NPT + kk * (NPT // 4), NPT // 4)])
    plsc.subcore_barrier()

    @pl.loop(0, CPW)
    def _(i):
        base = pl.multiple_of(wid * EPW + i * CHN, CHN)
        pltpu.sync_copy(row_hbm.at[pl.ds(base, CHN)], rowbuf)
        pltpu.sync_copy(col_hbm.at[pl.ds(base, CHN)], colbuf)
        pltpu.sync_copy(w_hbm.at[pl.ds(base, CHN)], wbuf)
        pltpu.async_copy(x1_hbm.at[rowbuf], msg, sem).wait()

        @pl.loop(0, CHN // 16)
        def _(g):
            wv = wbuf[pl.ds(g * 16, 16)]
            for j in range(16):
                wvb = lax.broadcast_in_dim(wv[j], (16,), ())
                for q in range(H // 16):
                    sl = (g * 16 + j, pl.ds(q * 16, 16))
                    msg[sl] = msg[sl] * wvb

        pltpu.sync_copy(msg, acc.at[colbuf], add=True)

    plsc.subcore_barrier()
    my_rows = pl.ds(sid * NPT, NPT)
    pltpu.sync_copy(acc.at[my_rows], out_hbm.at[cid, my_rows])


@functools.partial(
    pl.kernel,
    out_type=jax.ShapeDtypeStruct((NC, NP, C48), jnp.float32),
    mesh=_mesh,
    compiler_params=pltpu.CompilerParams(use_tc_tiling_on_sc=False),
    scratch_types=[
        pltpu.VMEM((CHN,), jnp.int32),
        pltpu.VMEM((CHN,), jnp.int32),
        pltpu.VMEM((CHN, C48), jnp.float32),
        pltpu.VMEM((NPT // 4, C48), jnp.float32),
        pltpu.VMEM_SHARED((NP, C48), jnp.float32),
        pltpu.SemaphoreType.DMA,
    ],
)
def _sc_agg2(row_hbm, col_hbm, t_hbm, out_hbm,
             rowbuf, colbuf, msg, zb, acc, sem):
    cid = lax.axis_index("c")
    sid = lax.axis_index("s")
    wid = cid * NS + sid

    _zero_rows(zb, C48)
    for kk in range(4):
        pltpu.sync_copy(zb, acc.at[pl.ds(sid * NPT + kk * (NPT // 4), NPT // 4)])
    plsc.subcore_barrier()

    @pl.loop(0, CPW)
    def _(i):
        base = pl.multiple_of(wid * EPW + i * CHN, CHN)
        pltpu.sync_copy(row_hbm.at[pl.ds(base, CHN)], rowbuf)
        pltpu.sync_copy(col_hbm.at[pl.ds(base, CHN)], colbuf)
        pltpu.async_copy(t_hbm.at[rowbuf], msg, sem).wait()
        pltpu.sync_copy(msg, acc.at[colbuf], add=True)

    plsc.subcore_barrier()
    my_rows = pl.ds(sid * NPT, NPT)
    pltpu.sync_copy(acc.at[my_rows], out_hbm.at[cid, my_rows])


def _tc_x1_body(x_ref, w1_ref, deg1_ref, deg2_ref, x1_ref, d1_ref, d2_ref):
    d1 = lax.rsqrt(deg1_ref[...])
    d2 = lax.rsqrt(deg2_ref[...])
    xw = jnp.dot(x_ref[...], w1_ref[...], preferred_element_type=jnp.float32)
    x1_ref[...] = xw * d1
    d1_ref[...] = d1
    d2_ref[...] = d2


_tc_x1 = pl.pallas_call(
    _tc_x1_body,
    out_shape=[
        jax.ShapeDtypeStruct((NP, H), jnp.float32),
        jax.ShapeDtypeStruct((NP, 1), jnp.float32),
        jax.ShapeDtypeStruct((NP, 1), jnp.float32),
    ],
)


def _tc_mid_body(accp_ref, x1_ref, d1_ref, d2_ref, b1_ref, w2_ref, t_ref):
    a = accp_ref[0] + accp_ref[1] + x1_ref[...]
    h = jnp.maximum(d1_ref[...] * a + b1_ref[...], 0.0)
    t_ref[...] = jnp.dot(h, w2_ref[...], preferred_element_type=jnp.float32) * d2_ref[...]


_tc_mid = pl.pallas_call(
    _tc_mid_body,
    out_shape=jax.ShapeDtypeStruct((NP, C48), jnp.float32),
)


def _tc_out_body(accp_ref, t_ref, d2_ref, b2_ref, o_ref):
    lg = d2_ref[...] * (accp_ref[0] + accp_ref[1] + t_ref[...]) + b2_ref[...]
    mask = lax.broadcasted_iota(jnp.int32, (NP, C48), 1) < C
    l = jnp.where(mask, lg, -1e30)
    mx = jnp.max(l, axis=1, keepdims=True)
    s = jnp.sum(jnp.exp(l - mx), axis=1, keepdims=True)
    o_ref[...] = l - mx - jnp.log(s)


_tc_out = pl.pallas_call(
    _tc_out_body,
    out_shape=jax.ShapeDtypeStruct((NP, C48), jnp.float32),
)


def kernel(x, edge_index, edge_weight, W1, b1, W2, b2):
    f32 = jnp.float32
    row = edge_index[0]
    col = edge_index[1]

    npad = EP - E
    pad_dst = (N + (jnp.arange(npad, dtype=jnp.int32) % (NP - N))).astype(jnp.int32)
    rowp = jnp.concatenate([row, jnp.zeros((npad,), jnp.int32)])
    colp = jnp.concatenate([col, pad_dst])
    wp = jnp.concatenate([edge_weight.astype(f32), jnp.zeros((npad,), f32)])

    xp = jnp.zeros((NP, D), f32).at[:N].set(x.astype(f32))
    w2p = jnp.zeros((H, C48), f32).at[:, :C].set(W2.astype(f32))
    b2p = jnp.zeros((1, C48), f32).at[0, :C].set(b2.astype(f32))
    b1r = b1.astype(f32).reshape(1, H)

    degs = _sc_degrees(colp, wp)
    deg1 = (degs[0, 0] + degs[1, 0] + 1.0)[:, None]
    deg2 = (degs[0, 1] + degs[1, 1] + 1.0)[:, None]

    x1, d1, d2 = _tc_x1(xp, W1.astype(f32), deg1, deg2)
    acc1 = _sc_agg1(rowp, colp, wp, x1)
    t = _tc_mid(acc1, x1, d1, d2, b1r, w2p)
    acc2 = _sc_agg2(rowp, colp, t)
    o = _tc_out(acc2, t, d2, b2p)
    return o[:N, :C]
